# baseline (speedup 1.0000x reference)
import functools
import jax
import jax.numpy as jnp
from jax import lax
from jax.experimental import pallas as pl
from jax.experimental.pallas import tpu as pltpu
from jax.experimental.pallas import tpu_sc as plsc

N_IN = 65536
N_OUT = 16384
B = 256
LOGN = 11
NNZ = N_OUT * LOGN

NC = 2
NS = 16
NW = NC * NS
COLS_W = N_OUT // NW
NNZ_W = COLS_W * LOGN
G = 8
NNZ_G = G * LOGN
GROUPS = COLS_W // G
LANES = 16
W = B // 2
WCHUNKS = W // LANES


@functools.partial(
    pl.kernel,
    out_type=jax.ShapeDtypeStruct((N_OUT, W), jnp.int32),
    mesh=plsc.VectorSubcoreMesh(core_axis_name="c", subcore_axis_name="s"),
    compiler_params=pltpu.CompilerParams(needs_layout_passes=False),
    scratch_types=[
        pltpu.VMEM((NNZ_W,), jnp.int32),
        pltpu.VMEM((NNZ_W + LANES,), jnp.float32),
        pltpu.VMEM((2, NNZ_G, W), jnp.int32),
        pltpu.VMEM((G, W), jnp.int32),
        pltpu.SemaphoreType.DMA,
        pltpu.SemaphoreType.DMA,
    ],
)
def _sc_bag(xT_hbm, idx_hbm, vals_hbm, out_hbm,
            idx_v, vals_v, rows_v, acc_v, sem0, sem1):
    wid = lax.axis_index("s") * NC + lax.axis_index("c")
    nz_base = wid * NNZ_W
    col_base = wid * COLS_W
    sems = (sem0, sem1)

    pltpu.sync_copy(idx_hbm.at[pl.ds(nz_base, NNZ_W)], idx_v)
    pltpu.sync_copy(vals_hbm.at[pl.ds(nz_base, NNZ_W)], vals_v.at[pl.ds(0, NNZ_W)])

    def start_gather(g, buf):
        pltpu.async_copy(
            xT_hbm.at[idx_v.at[pl.ds(g * NNZ_G, NNZ_G)]],
            rows_v.at[buf], sems[buf])

    def wait_gather(buf):
        pltpu.make_async_copy(
            xT_hbm.at[idx_v.at[pl.ds(0, NNZ_G)]],
            rows_v.at[buf], sems[buf]).wait()

    start_gather(0, 0)
    start_gather(1, 1)

    @pl.loop(0, GROUPS, step=2)
    def _groups(g0):
        for bslot in range(2):
            g = g0 + bslot
            wait_gather(bslot)

            @plsc.parallel_loop(0, G)
            def _cols(j):
                nz0 = j * LOGN
                vvec = vals_v[pl.ds(g * NNZ_G + nz0, LANES)]
                vs = []
                for t in range(LOGN):
                    v16 = jnp.broadcast_to(vvec[t], (LANES,))
                    vs.append(plsc.pack(v16, v16,
                                        format=plsc.PackFormat.INTERLEAVED))
                for c in range(WCHUNKS):
                    sl = pl.ds(c * LANES, LANES)
                    terms = [
                        plsc.bitcast(rows_v[bslot, nz0 + t, sl],
                                     jnp.bfloat16) * vs[t]
                        for t in range(LOGN)
                    ]
                    while len(terms) > 1:
                        terms = ([terms[i] + terms[i + 1]
                                  for i in range(0, len(terms) - 1, 2)]
                                 + ([terms[-1]] if len(terms) % 2 else []))
                    acc_v[j, sl] = plsc.bitcast(terms[0], jnp.int32)

            pltpu.sync_copy(acc_v, out_hbm.at[pl.ds(col_base + g * G, G)])

            @pl.when(g + 2 < GROUPS)
            def _():
                start_gather(g + 2, bslot)


def kernel(x, values, b, indices_1, indices_2):
    xT = lax.bitcast_convert_type(
        x.T.astype(jnp.bfloat16).reshape(N_IN, W, 2), jnp.int32)
    yT = lax.bitcast_convert_type(
        _sc_bag(xT, indices_1, values), jnp.bfloat16).reshape(N_OUT, B)
    return yT.T.astype(jnp.float32) + b


# speedup vs baseline: 1.0011x; 1.0011x over previous
import functools
import jax
import jax.numpy as jnp
from jax import lax
from jax.experimental import pallas as pl
from jax.experimental.pallas import tpu as pltpu
from jax.experimental.pallas import tpu_sc as plsc

N_IN = 65536
N_OUT = 16384
B = 256
LOGN = 11
NNZ = N_OUT * LOGN

NC = 2
NS = 16
NW = NC * NS
COLS_W = N_OUT // NW
NNZ_W = COLS_W * LOGN
G = 8
NNZ_G = G * LOGN
GROUPS = COLS_W // G
LANES = 16
W = B // 2
WCHUNKS = W // LANES


@functools.partial(
    pl.kernel,
    out_type=jax.ShapeDtypeStruct((N_OUT, W), jnp.int32),
    mesh=plsc.VectorSubcoreMesh(core_axis_name="c", subcore_axis_name="s"),
    compiler_params=pltpu.CompilerParams(needs_layout_passes=False),
    scratch_types=[
        pltpu.VMEM((NNZ_W,), jnp.int32),
        pltpu.VMEM((NNZ_W + LANES,), jnp.float32),
        pltpu.VMEM((2, NNZ_G, W), jnp.int32),
        pltpu.VMEM((G, W), jnp.int32),
        pltpu.SemaphoreType.DMA,
        pltpu.SemaphoreType.DMA,
    ],
)
def _sc_bag(xT_hbm, idx_hbm, vals_hbm, out_hbm,
            idx_v, vals_v, rows_v, acc_v, sem0, sem1):
    wid = lax.axis_index("s") * NC + lax.axis_index("c")
    nz_base = wid * NNZ_W
    col_base = wid * COLS_W
    sems = (sem0, sem1)

    pltpu.sync_copy(idx_hbm.at[pl.ds(nz_base, NNZ_W)], idx_v)
    pltpu.sync_copy(vals_hbm.at[pl.ds(nz_base, NNZ_W)], vals_v.at[pl.ds(0, NNZ_W)])

    def start_gather(g, buf):
        pltpu.async_copy(
            xT_hbm.at[idx_v.at[pl.ds(g * NNZ_G, NNZ_G)]],
            rows_v.at[buf], sems[buf])

    def wait_gather(buf):
        pltpu.make_async_copy(
            xT_hbm.at[idx_v.at[pl.ds(0, NNZ_G)]],
            rows_v.at[buf], sems[buf]).wait()

    start_gather(0, 0)
    start_gather(1, 1)

    @pl.loop(0, GROUPS, step=2)
    def _groups(g0):
        for bslot in range(2):
            g = g0 + bslot
            wait_gather(bslot)

            @plsc.parallel_loop(0, G)
            def _cols(j):
                nz0 = j * LOGN
                vvec = vals_v[pl.ds(g * NNZ_G + nz0, LANES)]
                vs = []
                for t in range(LOGN):
                    v16 = jnp.broadcast_to(vvec[t], (LANES,))
                    vs.append(plsc.pack(v16, v16,
                                        format=plsc.PackFormat.INTERLEAVED))
                for c in range(WCHUNKS):
                    sl = pl.ds(c * LANES, LANES)
                    terms = [
                        plsc.bitcast(rows_v[bslot, nz0 + t, sl],
                                     jnp.bfloat16) * vs[t]
                        for t in range(LOGN)
                    ]
                    while len(terms) > 1:
                        terms = ([terms[i] + terms[i + 1]
                                  for i in range(0, len(terms) - 1, 2)]
                                 + ([terms[-1]] if len(terms) % 2 else []))
                    acc_v[j, sl] = plsc.bitcast(terms[0], jnp.int32)

            pltpu.sync_copy(acc_v, out_hbm.at[pl.ds(col_base + g * G, G)])

            @pl.when(g + 2 < GROUPS)
            def _():
                start_gather(g + 2, bslot)


def kernel(x, values, b, indices_1, indices_2):
    xb = lax.optimization_barrier(
        lax.bitcast_convert_type(
            x.astype(jnp.bfloat16).reshape(W, 2, N_IN).swapaxes(1, 2),
            jnp.int32))
    xT = xb.T
    yT = lax.bitcast_convert_type(
        _sc_bag(xT, indices_1, values), jnp.bfloat16).reshape(N_OUT, B)
    return yT.T.astype(jnp.float32) + b


# speedup vs baseline: 2.1812x; 2.1788x over previous
import functools
import jax
import jax.numpy as jnp
from jax import lax
from jax.experimental import pallas as pl
from jax.experimental.pallas import tpu as pltpu
from jax.experimental.pallas import tpu_sc as plsc

N_IN = 65536
N_OUT = 16384
B = 256
LOGN = 11
NNZ = N_OUT * LOGN

NC = 2
NS = 16
NW = NC * NS
COLS_W = N_OUT // NW
NNZ_W = COLS_W * LOGN
G = 8
NNZ_G = G * LOGN
GROUPS = COLS_W // G
LANES = 16
W = B // 2
WCHUNKS = W // LANES

ROWS_W = N_IN // NW
PCH = 128
PCHUNKS = ROWS_W // PCH


@functools.partial(
    pl.kernel,
    out_type=jax.ShapeDtypeStruct((N_IN, W), jnp.int32),
    mesh=plsc.VectorSubcoreMesh(core_axis_name="c", subcore_axis_name="s"),
    compiler_params=pltpu.CompilerParams(needs_layout_passes=False),
    scratch_types=[
        pltpu.VMEM((2, PCH, B), jnp.float32),
        pltpu.VMEM((2, PCH, W), jnp.int32),
        pltpu.SemaphoreType.DMA,
        pltpu.SemaphoreType.DMA,
        pltpu.SemaphoreType.DMA,
        pltpu.SemaphoreType.DMA,
    ],
)
def _sc_pack(xT_hbm, out_hbm, in_v, out_v, si0, si1, so0, so1):
    wid = lax.axis_index("s") * NC + lax.axis_index("c")
    row_base = wid * ROWS_W
    sin = (si0, si1)
    sout = (so0, so1)

    def start_in(ch, buf):
        pltpu.async_copy(xT_hbm.at[pl.ds(row_base + ch * PCH, PCH)],
                         in_v.at[buf], sin[buf])

    def wait_in(buf):
        pltpu.make_async_copy(xT_hbm.at[pl.ds(0, PCH)], in_v.at[buf],
                              sin[buf]).wait()

    def start_out(ch, buf):
        pltpu.async_copy(out_v.at[buf],
                         out_hbm.at[pl.ds(row_base + ch * PCH, PCH)],
                         sout[buf])

    def wait_out(buf):
        pltpu.make_async_copy(out_v.at[buf], out_hbm.at[pl.ds(0, PCH)],
                              sout[buf]).wait()

    start_in(0, 0)
    start_in(1, 1)

    @pl.loop(0, PCHUNKS, step=2)
    def _chunks(ch0):
        for buf in range(2):
            ch = ch0 + buf
            wait_in(buf)

            @pl.when(ch >= 2)
            def _():
                wait_out(buf)

            @plsc.parallel_loop(0, PCH)
            def _rows(r):
                for c in range(WCHUNKS):
                    a = in_v[buf, r, pl.ds(c * 2 * LANES, LANES)]
                    bb = in_v[buf, r, pl.ds(c * 2 * LANES + LANES, LANES)]
                    packed = plsc.pack(a, bb,
                                       format=plsc.PackFormat.INTERLEAVED)
                    out_v[buf, r, pl.ds(c * LANES, LANES)] = plsc.bitcast(
                        packed, jnp.int32)

            start_out(ch, buf)

            @pl.when(ch + 2 < PCHUNKS)
            def _():
                start_in(ch + 2, buf)

    wait_out(0)
    wait_out(1)


@functools.partial(
    pl.kernel,
    out_type=jax.ShapeDtypeStruct((N_OUT, W), jnp.int32),
    mesh=plsc.VectorSubcoreMesh(core_axis_name="c", subcore_axis_name="s"),
    compiler_params=pltpu.CompilerParams(needs_layout_passes=False),
    scratch_types=[
        pltpu.VMEM((NNZ_W,), jnp.int32),
        pltpu.VMEM((NNZ_W + LANES,), jnp.float32),
        pltpu.VMEM((2, NNZ_G, W), jnp.int32),
        pltpu.VMEM((G, W), jnp.int32),
        pltpu.SemaphoreType.DMA,
        pltpu.SemaphoreType.DMA,
    ],
)
def _sc_bag(xP_hbm, idx_hbm, vals_hbm, out_hbm,
            idx_v, vals_v, rows_v, acc_v, sem0, sem1):
    wid = lax.axis_index("s") * NC + lax.axis_index("c")
    nz_base = wid * NNZ_W
    col_base = wid * COLS_W
    sems = (sem0, sem1)

    pltpu.sync_copy(idx_hbm.at[pl.ds(nz_base, NNZ_W)], idx_v)
    pltpu.sync_copy(vals_hbm.at[pl.ds(nz_base, NNZ_W)], vals_v.at[pl.ds(0, NNZ_W)])

    def start_gather(g, buf):
        pltpu.async_copy(
            xP_hbm.at[idx_v.at[pl.ds(g * NNZ_G, NNZ_G)]],
            rows_v.at[buf], sems[buf])

    def wait_gather(buf):
        pltpu.make_async_copy(
            xP_hbm.at[idx_v.at[pl.ds(0, NNZ_G)]],
            rows_v.at[buf], sems[buf]).wait()

    start_gather(0, 0)
    start_gather(1, 1)

    @pl.loop(0, GROUPS, step=2)
    def _groups(g0):
        for bslot in range(2):
            g = g0 + bslot
            wait_gather(bslot)

            @plsc.parallel_loop(0, G)
            def _cols(j):
                nz0 = j * LOGN
                vvec = vals_v[pl.ds(g * NNZ_G + nz0, LANES)]
                vs = []
                for t in range(LOGN):
                    v16 = jnp.broadcast_to(vvec[t], (LANES,))
                    vs.append(plsc.pack(v16, v16,
                                        format=plsc.PackFormat.INTERLEAVED))
                for c in range(WCHUNKS):
                    sl = pl.ds(c * LANES, LANES)
                    terms = [
                        plsc.bitcast(rows_v[bslot, nz0 + t, sl],
                                     jnp.bfloat16) * vs[t]
                        for t in range(LOGN)
                    ]
                    while len(terms) > 1:
                        terms = ([terms[i] + terms[i + 1]
                                  for i in range(0, len(terms) - 1, 2)]
                                 + ([terms[-1]] if len(terms) % 2 else []))
                    acc_v[j, sl] = plsc.bitcast(terms[0], jnp.int32)

            pltpu.sync_copy(acc_v, out_hbm.at[pl.ds(col_base + g * G, G)])

            @pl.when(g + 2 < GROUPS)
            def _():
                start_gather(g + 2, bslot)


def kernel(x, values, b, indices_1, indices_2):
    xT = x.T
    xP = _sc_pack(xT)
    yP = lax.bitcast_convert_type(
        _sc_bag(xP, indices_1, values), jnp.bfloat16)
    y = yP.reshape(N_OUT, WCHUNKS, LANES, 2).transpose(0, 1, 3, 2)
    y = y.reshape(N_OUT, B).T.astype(jnp.float32) + b
    return y
